# baseline (speedup 1.0000x reference)
import jax
import jax.numpy as jnp
from jax.experimental import pallas as pl
from jax.experimental.pallas import tpu as pltpu

_NUM_EXPERTS = 16
_TOP_K = 8
_HIDDEN = 768
_INNER = _HIDDEN * 4


def _router_kernel(x_ref, rw_ref, rb_ref, logits_ref, comb_ref):
    x = x_ref[...]
    logits = jnp.dot(x, rw_ref[...], preferred_element_type=jnp.float32)
    logits = logits + rb_ref[...]
    logits_ref[...] = logits
    m = jnp.max(logits, axis=-1, keepdims=True)
    ex = jnp.exp(logits - m)
    probs = ex / jnp.sum(ex, axis=-1, keepdims=True)
    t, e = probs.shape
    col = jax.lax.broadcasted_iota(jnp.int32, (t, e), 1)
    rank = jnp.zeros((t, e), jnp.int32)
    for j in range(e):
        pj = probs[:, j:j + 1]
        beats = (pj > probs) | ((pj == probs) & (j < col))
        rank = rank + beats.astype(jnp.int32)
    comb_ref[...] = jnp.where(rank < _TOP_K, probs, 0.0)


def _ffn_kernel(x_ref, pw_ref, pb_ref, ow_ref, ob_ref, w_ref, out_ref,
                h_ref, act_ref):
    s = pl.program_id(0)
    H = _HIDDEN
    I = _INNER
    n_proj = 2 * _NUM_EXPERTS

    @pl.when(s == 0)
    def _init():
        out_ref[...] = jnp.zeros_like(out_ref)

    @pl.when((s >= 2) & (s % 2 == 0))
    def _out_lo():
        y = jnp.dot(act_ref[:, :I // 2], ow_ref[0],
                    preferred_element_type=jnp.float32)
        w = w_ref[0, 0, :][:, None]
        out_ref[...] += y * w + w * ob_ref[0, 0, :][None, :]

    @pl.when((s >= 2) & (s % 2 == 1))
    def _out_hi():
        y = jnp.dot(act_ref[:, I // 2:], ow_ref[0],
                    preferred_element_type=jnp.float32)
        w = w_ref[0, 0, :][:, None]
        out_ref[...] += y * w

    @pl.when((s < n_proj) & (s % 2 == 0))
    def _proj_lo():
        h_ref[...] = pb_ref[0] + jnp.dot(
            x_ref[:, :H // 2], pw_ref[0], preferred_element_type=jnp.float32)

    @pl.when((s < n_proj) & (s % 2 == 1))
    def _proj_hi():
        h = h_ref[...] + jnp.dot(
            x_ref[:, H // 2:], pw_ref[0], preferred_element_type=jnp.float32)
        a = h[:, :I]
        g = h[:, I:]
        act_ref[...] = a * (g * jax.nn.sigmoid(g))


def kernel(hidden_states, router_W, router_b, proj_W, proj_b, out_W, out_b):
    B, S, H = hidden_states.shape
    T = B * S
    E = _NUM_EXPERTS
    I = _INNER

    x = hidden_states.reshape(T, H)

    logits, comb = pl.pallas_call(
        _router_kernel,
        out_shape=(
            jax.ShapeDtypeStruct((T, E), jnp.float32),
            jax.ShapeDtypeStruct((T, E), jnp.float32),
        ),
    )(x, router_W, router_b.reshape(1, E))

    comb_t = comb.T.reshape(E, 1, T)
    proj_b3 = proj_b.reshape(E, 1, 2 * I)
    out_b3 = out_b.reshape(E, 1, H)

    n_steps = 2 * E + 2

    out = pl.pallas_call(
        _ffn_kernel,
        grid=(n_steps,),
        in_specs=[
            pl.BlockSpec((T, H), lambda s: (0, 0)),
            pl.BlockSpec(
                (1, H // 2, 2 * I),
                lambda s: (jnp.minimum(s, 2 * E - 1) // 2,
                           jnp.minimum(s, 2 * E - 1) % 2, 0)),
            pl.BlockSpec(
                (1, 1, 2 * I),
                lambda s: (jnp.minimum(s, 2 * E - 1) // 2, 0, 0)),
            pl.BlockSpec(
                (1, I // 2, H),
                lambda s: (jnp.maximum(s - 2, 0) // 2,
                           jnp.maximum(s - 2, 0) % 2, 0)),
            pl.BlockSpec(
                (1, 1, H),
                lambda s: (jnp.maximum(s - 2, 0) // 2, 0, 0)),
            pl.BlockSpec(
                (1, 1, T),
                lambda s: (jnp.maximum(s - 2, 0) // 2, 0, 0)),
        ],
        out_specs=pl.BlockSpec((T, H), lambda s: (0, 0)),
        out_shape=jax.ShapeDtypeStruct((T, H), jnp.float32),
        scratch_shapes=[
            pltpu.VMEM((T, 2 * I), jnp.float32),
            pltpu.VMEM((T, I), jnp.float32),
        ],
        compiler_params=pltpu.CompilerParams(
            dimension_semantics=("arbitrary",),
        ),
    )(x, proj_W, proj_b3, out_W, out_b3, comb_t)

    return out.reshape(B, S, H), logits.reshape(B, S, E)


# speedup vs baseline: 1.0041x; 1.0041x over previous
import jax
import jax.numpy as jnp
from jax.experimental import pallas as pl
from jax.experimental.pallas import tpu as pltpu

_NUM_EXPERTS = 16
_TOP_K = 8
_HIDDEN = 768
_INNER = _HIDDEN * 4


def _router_kernel(x_ref, rw_ref, rb_ref, logits_ref, comb_ref):
    x = x_ref[...]
    logits = jnp.dot(x, rw_ref[...], preferred_element_type=jnp.float32)
    logits = logits + rb_ref[...]
    logits_ref[...] = logits
    m = jnp.max(logits, axis=-1, keepdims=True)
    ex = jnp.exp(logits - m)
    probs = ex / jnp.sum(ex, axis=-1, keepdims=True)
    t, e = probs.shape
    col = jax.lax.broadcasted_iota(jnp.int32, (t, e), 1)
    rank = jnp.zeros((t, e), jnp.int32)
    for j in range(e):
        pj = probs[:, j:j + 1]
        beats = (pj > probs) | ((pj == probs) & (j < col))
        rank = rank + beats.astype(jnp.int32)
    comb_ref[...] = jnp.where(rank < _TOP_K, probs, 0.0)


def _ffn_kernel(x_ref, pw_ref, pb_ref, ow_ref, ob_ref, w_ref, out_ref,
                h0_ref, h1_ref):
    s = pl.program_id(0)
    H = _HIDDEN
    I = _INNER
    n_proj = 2 * _NUM_EXPERTS
    cur_even = (s // 2) % 2 == 0

    @pl.when(s == 0)
    def _init():
        out_ref[...] = jnp.zeros_like(out_ref)

    def _out_lo(h_ref):
        h = h_ref
        a = h[:, :I // 2]
        g = h[:, I:I + I // 2]
        act = a * (g * jax.nn.sigmoid(g))
        y = jnp.dot(act, ow_ref[0], preferred_element_type=jnp.float32)
        w = w_ref[0, 0, :][:, None]
        out_ref[...] += y * w + w * ob_ref[0, 0, :][None, :]

    def _out_hi(h_ref):
        h = h_ref
        a = h[:, I // 2:I]
        g = h[:, I + I // 2:]
        act = a * (g * jax.nn.sigmoid(g))
        y = jnp.dot(act, ow_ref[0], preferred_element_type=jnp.float32)
        w = w_ref[0, 0, :][:, None]
        out_ref[...] += y * w

    @pl.when((s >= 2) & (s % 2 == 0) & cur_even)
    def _():
        _out_lo(h1_ref[...])

    @pl.when((s >= 2) & (s % 2 == 0) & jnp.logical_not(cur_even))
    def _():
        _out_lo(h0_ref[...])

    @pl.when((s >= 2) & (s % 2 == 1) & cur_even)
    def _():
        _out_hi(h1_ref[...])

    @pl.when((s >= 2) & (s % 2 == 1) & jnp.logical_not(cur_even))
    def _():
        _out_hi(h0_ref[...])

    def _proj_lo(h_ref):
        h_ref[...] = pb_ref[0] + jnp.dot(
            x_ref[:, :H // 2], pw_ref[0], preferred_element_type=jnp.float32)

    def _proj_hi(h_ref):
        h_ref[...] += jnp.dot(
            x_ref[:, H // 2:], pw_ref[0], preferred_element_type=jnp.float32)

    @pl.when((s < n_proj) & (s % 2 == 0) & cur_even)
    def _():
        _proj_lo(h0_ref)

    @pl.when((s < n_proj) & (s % 2 == 0) & jnp.logical_not(cur_even))
    def _():
        _proj_lo(h1_ref)

    @pl.when((s < n_proj) & (s % 2 == 1) & cur_even)
    def _():
        _proj_hi(h0_ref)

    @pl.when((s < n_proj) & (s % 2 == 1) & jnp.logical_not(cur_even))
    def _():
        _proj_hi(h1_ref)


def kernel(hidden_states, router_W, router_b, proj_W, proj_b, out_W, out_b):
    B, S, H = hidden_states.shape
    T = B * S
    E = _NUM_EXPERTS
    I = _INNER

    x = hidden_states.reshape(T, H)

    logits, comb = pl.pallas_call(
        _router_kernel,
        out_shape=(
            jax.ShapeDtypeStruct((T, E), jnp.float32),
            jax.ShapeDtypeStruct((T, E), jnp.float32),
        ),
    )(x, router_W, router_b.reshape(1, E))

    comb_t = comb.T.reshape(E, 1, T)
    proj_b3 = proj_b.reshape(E, 1, 2 * I)
    out_b3 = out_b.reshape(E, 1, H)

    n_steps = 2 * E + 2

    out = pl.pallas_call(
        _ffn_kernel,
        grid=(n_steps,),
        in_specs=[
            pl.BlockSpec((T, H), lambda s: (0, 0)),
            pl.BlockSpec(
                (1, H // 2, 2 * I),
                lambda s: (jnp.minimum(s, 2 * E - 1) // 2,
                           jnp.minimum(s, 2 * E - 1) % 2, 0)),
            pl.BlockSpec(
                (1, 1, 2 * I),
                lambda s: (jnp.minimum(s, 2 * E - 1) // 2, 0, 0)),
            pl.BlockSpec(
                (1, I // 2, H),
                lambda s: (jnp.maximum(s - 2, 0) // 2,
                           jnp.maximum(s - 2, 0) % 2, 0)),
            pl.BlockSpec(
                (1, 1, H),
                lambda s: (jnp.maximum(s - 2, 0) // 2, 0, 0)),
            pl.BlockSpec(
                (1, 1, T),
                lambda s: (jnp.maximum(s - 2, 0) // 2, 0, 0)),
        ],
        out_specs=pl.BlockSpec((T, H), lambda s: (0, 0)),
        out_shape=jax.ShapeDtypeStruct((T, H), jnp.float32),
        scratch_shapes=[
            pltpu.VMEM((T, 2 * I), jnp.float32),
            pltpu.VMEM((T, 2 * I), jnp.float32),
        ],
        compiler_params=pltpu.CompilerParams(
            dimension_semantics=("arbitrary",),
        ),
    )(x, proj_W, proj_b3, out_W, out_b3, comb_t)

    return out.reshape(B, S, H), logits.reshape(B, S, E)


# speedup vs baseline: 1.0368x; 1.0326x over previous
import jax
import jax.numpy as jnp
from jax.experimental import pallas as pl
from jax.experimental.pallas import tpu as pltpu

_NUM_EXPERTS = 16
_TOP_K = 8
_HIDDEN = 768
_INNER = _HIDDEN * 4
_CHUNK = 1536


def _moe_kernel(x_ref, rw_ref, rb_ref, pa_ref, pg_ref, pba_ref, pbg_ref,
                ow_ref, ob_ref, out_ref, logits_ref, comb_ref):
    e = pl.program_id(0)
    c = pl.program_id(1)

    @pl.when((e == 0) & (c == 0))
    def _router_and_init():
        x = x_ref[...]
        logits = jnp.dot(x, rw_ref[...], preferred_element_type=jnp.float32)
        logits = logits + rb_ref[...]
        logits_ref[...] = logits
        m = jnp.max(logits, axis=-1, keepdims=True)
        ex = jnp.exp(logits - m)
        probs = ex / jnp.sum(ex, axis=-1, keepdims=True)
        t, ne = probs.shape
        col = jax.lax.broadcasted_iota(jnp.int32, (t, ne), 1)
        rank = jnp.zeros((t, ne), jnp.int32)
        for j in range(ne):
            pj = probs[:, j:j + 1]
            beats = (pj > probs) | ((pj == probs) & (j < col))
            rank = rank + beats.astype(jnp.int32)
        comb_ref[...] = jnp.where(rank < _TOP_K, probs, 0.0)
        out_ref[...] = jnp.zeros_like(out_ref)

    x = x_ref[...]
    a = jnp.dot(x, pa_ref[0], preferred_element_type=jnp.float32)
    a = a + pba_ref[0, 0, :][None, :]
    g = jnp.dot(x, pg_ref[0], preferred_element_type=jnp.float32)
    g = g + pbg_ref[0, 0, :][None, :]
    act = a * (g * jax.nn.sigmoid(g))
    y = jnp.dot(act, ow_ref[0], preferred_element_type=jnp.float32)

    comb = comb_ref[...]
    lane = jax.lax.broadcasted_iota(jnp.int32, comb.shape, 1)
    w = jnp.sum(jnp.where(lane == e, comb, 0.0), axis=1)[:, None]
    contrib = y * w

    @pl.when(c == 0)
    def _bias():
        out_ref[...] += w * ob_ref[0, 0, :][None, :]

    out_ref[...] += contrib


def kernel(hidden_states, router_W, router_b, proj_W, proj_b, out_W, out_b):
    B, S, H = hidden_states.shape
    T = B * S
    E = _NUM_EXPERTS
    I = _INNER
    C = _CHUNK
    NC = I // C

    x = hidden_states.reshape(T, H)
    proj_b3 = proj_b.reshape(E, 1, 2 * I)
    out_b3 = out_b.reshape(E, 1, H)

    out, logits = pl.pallas_call(
        _moe_kernel,
        grid=(E, NC),
        in_specs=[
            pl.BlockSpec((T, H), lambda e, c: (0, 0)),
            pl.BlockSpec((H, E), lambda e, c: (0, 0)),
            pl.BlockSpec((1, E), lambda e, c: (0, 0)),
            pl.BlockSpec((1, H, C), lambda e, c: (e, 0, c)),
            pl.BlockSpec((1, H, C), lambda e, c: (e, 0, c + NC)),
            pl.BlockSpec((1, 1, C), lambda e, c: (e, 0, c)),
            pl.BlockSpec((1, 1, C), lambda e, c: (e, 0, c + NC)),
            pl.BlockSpec((1, C, H), lambda e, c: (e, c, 0)),
            pl.BlockSpec((1, 1, H), lambda e, c: (e, 0, 0)),
        ],
        out_specs=(
            pl.BlockSpec((T, H), lambda e, c: (0, 0)),
            pl.BlockSpec((T, E), lambda e, c: (0, 0)),
        ),
        out_shape=(
            jax.ShapeDtypeStruct((T, H), jnp.float32),
            jax.ShapeDtypeStruct((T, E), jnp.float32),
        ),
        scratch_shapes=[
            pltpu.VMEM((T, E), jnp.float32),
        ],
        compiler_params=pltpu.CompilerParams(
            dimension_semantics=("arbitrary", "arbitrary"),
        ),
    )(x, router_W, router_b.reshape(1, E), proj_W, proj_W, proj_b3, proj_b3,
      out_W, out_b3)

    return out.reshape(B, S, H), logits.reshape(B, S, E)
